# baseline (speedup 1.0000x reference)
import functools

import jax
import jax.numpy as jnp
from jax import lax
from jax.experimental import pallas as pl
from jax.experimental.pallas import tpu as pltpu
from jax.experimental.pallas import tpu_sc as plsc

_B = 16384
_D = 64
_NU = 1000000
_NC = 2
_NS = 16
_NW = _NC * _NS
_BPW = _B // _NW

_CW = 256
_NCH = 3907
_JT = 123
_HCAP = 1024
_CCAP = 128
_SLOTS = _HCAP + _CCAP
_NR = _NW * _SLOTS
_IDUMP = _B + 8


def _sweep_gather(U, gmfT, mlpT, tailg, tailm):
    mesh = plsc.VectorSubcoreMesh(core_axis_name="c", subcore_axis_name="s")

    @functools.partial(
        pl.kernel,
        mesh=mesh,
        compiler_params=pltpu.CompilerParams(needs_layout_passes=False),
        out_type=(
            jax.ShapeDtypeStruct((_NR * 2 * _D,), jnp.float32),
            jax.ShapeDtypeStruct((_B + 16,), jnp.int32),
        ),
        scratch_types=[
            pltpu.VMEM((2048,), jnp.int32),
            pltpu.VMEM((_HCAP + 16,), jnp.int32),
            pltpu.VMEM((_HCAP + 16,), jnp.int32),
            pltpu.VMEM((_CCAP + 16,), jnp.int32),
            pltpu.VMEM((_CCAP + 16,), jnp.int32),
            pltpu.VMEM((_CCAP + 16,), jnp.int32),
            pltpu.VMEM((_D, _CW), jnp.float32),
            pltpu.VMEM((_D, _CW), jnp.float32),
            pltpu.VMEM((_CCAP * 2 * _D,), jnp.float32),
            pltpu.VMEM((32 * 2 * _D,), jnp.float32),
            pltpu.SemaphoreType.DMA,
            pltpu.SemaphoreType.DMA,
        ],
    )
    def k(u_hbm, g_hbm, m_hbm, tg_hbm, tm_hbm, rows_out, inv_out,
          u_sec, hit_u, hit_pos, cu, cpos, slotv, buf_g, buf_m, stage, dummy,
          sem_in, sem_out):
        w = lax.axis_index("s") * _NC + lax.axis_index("c")
        lanes = lax.iota(jnp.int32, 16)
        slot0 = w * _SLOTS

        for t in range(0, _HCAP + 16, 16):
            hit_u[pl.ds(t, 16)] = jnp.full((16,), -1, jnp.int32)

        nh = jnp.int32(0)
        for sec in range(_B // 2048):
            pltpu.async_copy(u_hbm.at[pl.ds(sec * 2048, 2048)], u_sec, sem_in).wait()

            def bucket_body(kk, nh):
                v = u_sec[pl.ds(kk * 16, 16)]
                m = ((v >> 8) & (_NW - 1)) == w
                mi = m.astype(jnp.int32)
                off = jnp.minimum(nh, _HCAP)
                idx = jnp.where(m, off + plsc.cumsum(mi) - 1, _HCAP + 8)
                plsc.store_scatter(hit_u, [idx], v)
                plsc.store_scatter(hit_pos, [idx],
                                   lanes + (sec * 2048 + kk * 16))
                return nh + jnp.sum(mi)

            nh = lax.fori_loop(0, 2048 // 16, bucket_body, nh)
        nh = jnp.minimum(nh, _HCAP)
        ng = (nh + 15) // 16

        def chunk_body(j, tot):
            c = w + _NW * j

            def do_chunk(tot):
                u0 = jnp.where(c == _NCH - 1, _NU - 128, c * _CW)

                @pl.when(c < _NCH - 1)
                def _():
                    hs = []
                    for r in range(8):
                        hs.append(pltpu.async_copy(
                            g_hbm.at[pl.ds(r * 8, 8), pl.ds(c * _CW, _CW)],
                            buf_g.at[pl.ds(r * 8, 8)], sem_in))
                        hs.append(pltpu.async_copy(
                            m_hbm.at[pl.ds(r * 8, 8), pl.ds(c * _CW, _CW)],
                            buf_m.at[pl.ds(r * 8, 8)], sem_in))
                    for h in hs:
                        h.wait()

                @pl.when(c == _NCH - 1)
                def _():
                    pltpu.sync_copy(tg_hbm, buf_g.at[:, pl.ds(0, 128)])
                    pltpu.sync_copy(tm_hbm, buf_m.at[:, pl.ds(0, 128)])

                for t in range(0, _CCAP + 16, 16):
                    cpos[pl.ds(t, 16)] = jnp.full((16,), _IDUMP, jnp.int32)

                def compact_body(g, nc):
                    hv = hit_u[pl.ds(g * 16, 16)]
                    pv = hit_pos[pl.ds(g * 16, 16)]
                    m = (hv >> 8) == c
                    mi = m.astype(jnp.int32)
                    off = jnp.minimum(nc, _CCAP)
                    idx = jnp.where(m, off + plsc.cumsum(mi) - 1, _CCAP + 8)
                    plsc.store_scatter(cu, [idx], hv)
                    plsc.store_scatter(cpos, [idx], pv)
                    return nc + jnp.sum(mi)

                nc = lax.fori_loop(0, ng, compact_body, jnp.int32(0))
                nc = jnp.minimum(nc, _CCAP)

                for t in range(0, _CCAP + 16, 16):
                    slotv[pl.ds(t, 16)] = lanes + (slot0 + tot + t)

                pass

                def extract_body(e, x):
                    uv = cu[pl.ds(e * 16, 16)]
                    duv = jnp.clip(uv - u0, 0, _CW - 1)
                    rowv = (e * 16 + lanes) * (2 * _D)
                    for f in range(_D):
                        fv = jnp.full((16,), f, jnp.int32)
                        gv = plsc.load_gather(buf_g, [fv, duv])
                        plsc.store_scatter(stage, [rowv + f], gv)
                        mv = plsc.load_gather(buf_m, [fv, duv])
                        plsc.store_scatter(stage, [rowv + (_D + f)], mv)
                    return x

                lax.fori_loop(0, 0, extract_body, jnp.int32(0))

                pltpu.async_copy(slotv, inv_out.at[cpos], sem_out).wait()

                def flush_body(b, x):
                    pltpu.async_copy(
                        stage.at[pl.ds(b * (32 * 2 * _D), 32 * 2 * _D)],
                        rows_out.at[pl.ds((slot0 + tot + b * 32) * (2 * _D),
                                          32 * 2 * _D)],
                        sem_out)
                    return x

                nb = jnp.int32(0)
                lax.fori_loop(0, nb, flush_body, jnp.int32(0))

                pass
                return tot + nc

            return lax.cond(c < _NCH, do_chunk, lambda t: t, tot)

        lax.fori_loop(0, _JT, chunk_body, jnp.int32(0))

        pass

    return k(U, gmfT, mlpT, tailg, tailm)


def _unsort_gather(rows2d, inv):
    mesh = plsc.VectorSubcoreMesh(core_axis_name="c", subcore_axis_name="s")

    @functools.partial(
        pl.kernel,
        mesh=mesh,
        compiler_params=pltpu.CompilerParams(use_tc_tiling_on_sc=False),
        out_type=jax.ShapeDtypeStruct((_B, 2 * _D), jnp.float32),
        scratch_types=[
            pltpu.VMEM((_BPW,), jnp.int32),
            pltpu.VMEM((_BPW, 2 * _D), jnp.float32),
            pltpu.SemaphoreType.DMA,
        ],
    )
    def k(inv_hbm, rows_hbm, out, idx_v, rows_v, sem):
        wid = lax.axis_index("s") * _NC + lax.axis_index("c")
        base = wid * _BPW
        pltpu.sync_copy(inv_hbm.at[pl.ds(base, _BPW)], idx_v)
        pltpu.async_copy(rows_hbm.at[idx_v], rows_v, sem).wait()
        pltpu.sync_copy(rows_v, out.at[pl.ds(base, _BPW)])

    return k(inv, rows2d)


def _bias_gather(U, bias1d):
    mesh = plsc.VectorSubcoreMesh(core_axis_name="c", subcore_axis_name="s")

    @functools.partial(
        pl.kernel,
        mesh=mesh,
        compiler_params=pltpu.CompilerParams(use_tc_tiling_on_sc=False),
        out_type=jax.ShapeDtypeStruct((_B,), jnp.float32),
        scratch_types=[
            pltpu.VMEM((_BPW,), jnp.int32),
            pltpu.VMEM((_BPW,), jnp.float32),
            pltpu.SemaphoreType.DMA,
        ],
    )
    def k(u_hbm, b_hbm, out_b, idx_v, rows_b, sem_b):
        wid = lax.axis_index("s") * _NC + lax.axis_index("c")
        base = wid * _BPW
        pltpu.sync_copy(u_hbm.at[pl.ds(base, _BPW)], idx_v)
        pltpu.async_copy(b_hbm.at[idx_v], rows_b, sem_b).wait()
        pltpu.sync_copy(rows_b, out_b.at[pl.ds(base, _BPW)])

    return k(U, bias1d)


def _dense_body(e_ref, ugm_ref, ub_ref,
                wg_ref, bg_ref, wm_ref, bm_ref,
                w0a_ref, w0b_ref, b0_ref, w1_ref, b1_ref, w2_ref, b2_ref,
                wfg_ref, wfh_ref, wib_ref, c_ref, out_ref):
    e = e_ref[:]
    ug = ugm_ref[:, :_D]
    um = ugm_ref[:, _D:]
    item_g = jnp.dot(e, wg_ref[:], preferred_element_type=jnp.float32) + bg_ref[:]
    gmf = ug * item_g
    item_m = jnp.dot(e, wm_ref[:], preferred_element_type=jnp.float32) + bm_ref[:]
    h = (jnp.dot(um, w0a_ref[:], preferred_element_type=jnp.float32)
         + jnp.dot(item_m, w0b_ref[:], preferred_element_type=jnp.float32)
         + b0_ref[:])
    h = jnp.maximum(h, 0.0)
    h = jnp.maximum(jnp.dot(h, w1_ref[:], preferred_element_type=jnp.float32) + b1_ref[:], 0.0)
    h = jnp.maximum(jnp.dot(h, w2_ref[:], preferred_element_type=jnp.float32) + b2_ref[:], 0.0)
    pred = (jnp.sum(gmf * wfg_ref[:][None, :], axis=1)
            + jnp.sum(h * wfh_ref[:][None, :], axis=1)
            + jnp.sum(e * wib_ref[:][None, :], axis=1))
    pred = pred + ub_ref[:] + c_ref[0]
    out_ref[:] = jax.nn.sigmoid(pred)


def _tc_dense(E, ugm, ub, Wg, bg, Wm, bm, W0a, W0b, b0, W1t, b1, W2t, b2,
              wfg, wfh, wib, c):
    grid = 8
    r = _B // grid

    def row2(d):
        return pl.BlockSpec((r, d), lambda i: (i, 0))

    row1 = pl.BlockSpec((r,), lambda i: (i,))

    def full2(a):
        return pl.BlockSpec(a.shape, lambda i: (0, 0))

    def full1(a):
        return pl.BlockSpec(a.shape, lambda i: (0,))

    return pl.pallas_call(
        _dense_body,
        grid=(grid,),
        in_specs=[row2(_D), row2(2 * _D), row1,
                  full2(Wg), full1(bg), full2(Wm), full1(bm),
                  full2(W0a), full2(W0b), full1(b0),
                  full2(W1t), full1(b1), full2(W2t), full1(b2),
                  full1(wfg), full1(wfh), full1(wib), full1(c)],
        out_specs=row1,
        out_shape=jax.ShapeDtypeStruct((_B,), jnp.float32),
    )(E, ugm, ub, Wg, bg, Wm, bm, W0a, W0b, b0, W1t, b1, W2t, b2,
      wfg, wfh, wib, c)


def kernel(U, E, user_emb_gmf, user_bias_tab, W_item_gmf, b_item_gmf,
           W_item_bias, b_item_bias, user_emb_mlp, W_item_mlp, b_item_mlp,
           W_mlp0, b_mlp0, W_mlp1, b_mlp1, W_mlp2, b_mlp2, W_final, b_final):
    u32 = U.astype(jnp.int32)
    gT = user_emb_gmf.T
    mT = user_emb_mlp.T
    rows_flat, inv_full = _sweep_gather(
        u32, gT, mT, gT[:, _NU - 128:], mT[:, _NU - 128:])
    ugm = _unsort_gather(rows_flat.reshape(_NR, 2 * _D), inv_full[:_B])
    ub = _bias_gather(u32, user_bias_tab.reshape(-1))
    Wg = W_item_gmf.T
    Wm = W_item_mlp.T
    W0a = W_mlp0[:, :_D].T
    W0b = W_mlp0[:, _D:].T
    W1t = W_mlp1.T
    W2t = W_mlp2.T
    wfg = W_final[0, :_D]
    wfh = W_final[0, _D:]
    wib = W_item_bias[0]
    c = b_final + b_item_bias
    return _tc_dense(E, ugm, ub, Wg, b_item_gmf, Wm, b_item_mlp,
                     W0a, W0b, b_mlp0, W1t, b_mlp1, W2t, b_mlp2,
                     wfg, wfh, wib, c)


# speedup vs baseline: 71.8031x; 71.8031x over previous
import functools

import jax
import jax.numpy as jnp
from jax import lax
from jax.experimental import pallas as pl
from jax.experimental.pallas import tpu as pltpu
from jax.experimental.pallas import tpu_sc as plsc

_B = 16384
_D = 64
_NC = 2
_NS = 16
_NW = _NC * _NS
_BPW = _B // _NW


def _sc_gather(U, gmf_tab, mlp_tab, bias_tab):
    mesh = plsc.VectorSubcoreMesh(core_axis_name="c", subcore_axis_name="s")

    @functools.partial(
        pl.kernel,
        mesh=mesh,
        compiler_params=pltpu.CompilerParams(use_tc_tiling_on_sc=False),
        out_type=(
            jax.ShapeDtypeStruct((_B, _D), jnp.float32),
            jax.ShapeDtypeStruct((_B, _D), jnp.float32),
            jax.ShapeDtypeStruct((_B,), jnp.float32),
        ),
        scratch_types=[
            pltpu.VMEM((_BPW,), jnp.int32),
            pltpu.VMEM((_BPW, _D), jnp.float32),
            pltpu.VMEM((_BPW, _D), jnp.float32),
            pltpu.VMEM((_BPW,), jnp.float32),
            pltpu.SemaphoreType.DMA,
            pltpu.SemaphoreType.DMA,
            pltpu.SemaphoreType.DMA,
        ],
    )
    def k(u_hbm, g_hbm, m_hbm, b_hbm, out_g, out_m, out_b,
          idx_v, rows_g, rows_m, rows_b, sem_g, sem_m, sem_b):
        wid = lax.axis_index("s") * _NC + lax.axis_index("c")
        base = wid * _BPW
        pltpu.sync_copy(u_hbm.at[pl.ds(base, _BPW)], idx_v)
        cg = pltpu.async_copy(g_hbm.at[idx_v], rows_g, sem_g)
        cm = pltpu.async_copy(m_hbm.at[idx_v], rows_m, sem_m)
        cb = pltpu.async_copy(b_hbm.at[idx_v], rows_b, sem_b)
        cg.wait()
        pltpu.sync_copy(rows_g, out_g.at[pl.ds(base, _BPW)])
        cm.wait()
        pltpu.sync_copy(rows_m, out_m.at[pl.ds(base, _BPW)])
        cb.wait()
        pltpu.sync_copy(rows_b, out_b.at[pl.ds(base, _BPW)])

    return k(U, gmf_tab, mlp_tab, bias_tab.reshape(-1))


def _dense_body(e_ref, ug_ref, um_ref, ub_ref,
                wg_ref, bg_ref, wm_ref, bm_ref,
                w0a_ref, w0b_ref, b0_ref, w1_ref, b1_ref, w2_ref, b2_ref,
                wfg_ref, wfh_ref, wib_ref, c_ref, out_ref):
    e = e_ref[:]
    item_g = jnp.dot(e, wg_ref[:], preferred_element_type=jnp.float32) + bg_ref[:]
    gmf = ug_ref[:] * item_g
    item_m = jnp.dot(e, wm_ref[:], preferred_element_type=jnp.float32) + bm_ref[:]
    h = (jnp.dot(um_ref[:], w0a_ref[:], preferred_element_type=jnp.float32)
         + jnp.dot(item_m, w0b_ref[:], preferred_element_type=jnp.float32)
         + b0_ref[:])
    h = jnp.maximum(h, 0.0)
    h = jnp.maximum(jnp.dot(h, w1_ref[:], preferred_element_type=jnp.float32) + b1_ref[:], 0.0)
    h = jnp.maximum(jnp.dot(h, w2_ref[:], preferred_element_type=jnp.float32) + b2_ref[:], 0.0)
    pred = (jnp.sum(gmf * wfg_ref[:][None, :], axis=1)
            + jnp.sum(h * wfh_ref[:][None, :], axis=1)
            + jnp.sum(e * wib_ref[:][None, :], axis=1))
    pred = pred + ub_ref[:] + c_ref[0]
    out_ref[:] = jax.nn.sigmoid(pred)


def _tc_dense(E, ug, um, ub, Wg, bg, Wm, bm, W0a, W0b, b0, W1t, b1, W2t, b2,
              wfg, wfh, wib, c):
    grid = 8
    r = _B // grid

    def row2(d):
        return pl.BlockSpec((r, d), lambda i: (i, 0))

    row1 = pl.BlockSpec((r,), lambda i: (i,))

    def full2(a):
        return pl.BlockSpec(a.shape, lambda i: (0, 0))

    def full1(a):
        return pl.BlockSpec(a.shape, lambda i: (0,))

    return pl.pallas_call(
        _dense_body,
        grid=(grid,),
        in_specs=[row2(_D), row2(_D), row2(_D), row1,
                  full2(Wg), full1(bg), full2(Wm), full1(bm),
                  full2(W0a), full2(W0b), full1(b0),
                  full2(W1t), full1(b1), full2(W2t), full1(b2),
                  full1(wfg), full1(wfh), full1(wib), full1(c)],
        out_specs=row1,
        out_shape=jax.ShapeDtypeStruct((_B,), jnp.float32),
    )(E, ug, um, ub, Wg, bg, Wm, bm, W0a, W0b, b0, W1t, b1, W2t, b2,
      wfg, wfh, wib, c)


def kernel(U, E, user_emb_gmf, user_bias_tab, W_item_gmf, b_item_gmf,
           W_item_bias, b_item_bias, user_emb_mlp, W_item_mlp, b_item_mlp,
           W_mlp0, b_mlp0, W_mlp1, b_mlp1, W_mlp2, b_mlp2, W_final, b_final):
    u32 = U.astype(jnp.int32)
    ug, um, ub = _sc_gather(u32, user_emb_gmf, user_emb_mlp, user_bias_tab)
    Wg = W_item_gmf.T
    Wm = W_item_mlp.T
    W0a = W_mlp0[:, :_D].T
    W0b = W_mlp0[:, _D:].T
    W1t = W_mlp1.T
    W2t = W_mlp2.T
    wfg = W_final[0, :_D]
    wfh = W_final[0, _D:]
    wib = W_item_bias[0]
    c = b_final + b_item_bias
    return _tc_dense(E, ug, um, ub, Wg, b_item_gmf, Wm, b_item_mlp,
                     W0a, W0b, b_mlp0, W1t, b_mlp1, W2t, b_mlp2,
                     wfg, wfh, wib, c)
